# Initial kernel scaffold; baseline (speedup 1.0000x reference)
import jax
import jax.numpy as jnp
from jax.experimental import pallas as pl


def kernel(data, length, table, W, b):
    raise NotImplementedError("write your pallas kernel here")


# speedup vs baseline: 9.0742x; 9.0742x over previous
import functools

import jax
import jax.numpy as jnp
from jax import lax
from jax.experimental import pallas as pl
from jax.experimental.pallas import tpu as pltpu
from jax.experimental.pallas import tpu_sc as plsc

LANES = 16


@functools.partial(jax.jit, static_argnames=("nb", "half", "emb"))
def _sc_pooled_sums(data2, table, *, nb, half, emb):
    B = data2.shape[0]
    mesh = plsc.VectorSubcoreMesh(core_axis_name="c", subcore_axis_name="s")
    ncores = mesh.num_cores

    def body(data_hbm, table_hbm, out_hbm, idx_v, rows_v, out_v, sem):
        wid = lax.axis_index("s") * ncores + lax.axis_index("c")
        base = wid * nb
        pltpu.sync_copy(data_hbm.at[pl.ds(base, nb)], idx_v)

        def row_body(r, carry):
            cp0 = pltpu.async_copy(table_hbm.at[idx_v.at[r, 0]], rows_v.at[0], sem)
            cp1 = pltpu.async_copy(table_hbm.at[idx_v.at[r, 1]], rows_v.at[1], sem)
            cp0.wait()
            cp1.wait()

            def acc_body(i, acc):
                out = []
                for c in range(emb // LANES):
                    v = acc[c]
                    for j in range(2):
                        v = v + rows_v[j, i, pl.ds(c * LANES, LANES)]
                    out.append(v)
                return tuple(out)

            zero = jnp.zeros((LANES,), jnp.float32)
            acc = lax.fori_loop(0, half, acc_body, (zero,) * (emb // LANES))
            for c in range(emb // LANES):
                out_v[r, pl.ds(c * LANES, LANES)] = acc[c]
            return carry

        lax.fori_loop(0, nb, row_body, 0)
        pltpu.sync_copy(out_v, out_hbm.at[pl.ds(base, nb)])

    return pl.kernel(
        body,
        out_type=jax.ShapeDtypeStruct((B, emb), jnp.float32),
        mesh=mesh,
        scratch_types=[
            pltpu.VMEM((nb, 2, half), jnp.int32),
            pltpu.VMEM((2, half, emb), jnp.float32),
            pltpu.VMEM((nb, emb), jnp.float32),
            pltpu.SemaphoreType.DMA,
        ],
        compiler_params=pltpu.CompilerParams(use_tc_tiling_on_sc=False),
    )(data2, table)


def _tc_body(sums_ref, len_ref, w_ref, b_ref, out_ref):
    s = sums_ref[...]
    out = lax.dot_general(
        s, w_ref[...], (((1,), (1,)), ((), ())), preferred_element_type=jnp.float32
    )
    out_ref[...] = out / len_ref[...] + b_ref[...]


@jax.jit
def _tc_linear(sums, length_col, W, b_row):
    B = sums.shape[0]
    n_cls = W.shape[0]
    return pl.pallas_call(
        _tc_body,
        out_shape=jax.ShapeDtypeStruct((B, n_cls), jnp.float32),
    )(sums, length_col, W, b_row)


def kernel(data, length, table, W, b):
    B, L = data.shape
    emb = table.shape[1]
    half = L // 2
    assert L % 2 == 0 and half <= 128 and emb % LANES == 0
    nw = 32
    nb = B // nw
    data2 = data.astype(jnp.int32).reshape(B, 2, half)
    sums = _sc_pooled_sums(data2, table, nb=nb, half=half, emb=emb)
    length_col = length.astype(jnp.float32).reshape(B, 1)
    return _tc_linear(sums, length_col, W, b.reshape(1, -1))


# speedup vs baseline: 16.1232x; 1.7768x over previous
import functools

import jax
import jax.numpy as jnp
from jax import lax
from jax.experimental import pallas as pl
from jax.experimental.pallas import tpu as pltpu
from jax.experimental.pallas import tpu_sc as plsc

LANES = 16


@functools.partial(jax.jit, static_argnames=("nb", "half", "emb"))
def _sc_pooled_sums(data2, table, *, nb, half, emb):
    B = data2.shape[0]
    mesh = plsc.VectorSubcoreMesh(core_axis_name="c", subcore_axis_name="s")
    ncores = mesh.num_cores

    NBUF = 4

    def body(data_hbm, table_hbm, out_hbm, idx_v, rows_v, out_v, *sems):
        wid = lax.axis_index("s") * ncores + lax.axis_index("c")
        base = wid * nb
        pltpu.sync_copy(data_hbm.at[pl.ds(base, nb)], idx_v)

        def issue(r, slot):
            for j in range(2):
                pltpu.async_copy(
                    table_hbm.at[idx_v.at[r, j]], rows_v.at[slot, j], sems[slot]
                )

        def wait(r, slot):
            for j in range(2):
                pltpu.make_async_copy(
                    table_hbm.at[idx_v.at[r, j]], rows_v.at[slot, j], sems[slot]
                ).wait()

        def accumulate(r, slot):
            def acc_body(i, acc):
                out = []
                for c in range(emb // LANES):
                    v = acc[c]
                    for u in range(2):
                        for j in range(2):
                            v = v + rows_v[slot, j, 2 * i + u, pl.ds(c * LANES, LANES)]
                    out.append(v)
                return tuple(out)

            zero = jnp.zeros((LANES,), jnp.float32)
            acc = lax.fori_loop(0, half // 2, acc_body, (zero,) * (emb // LANES))
            for c in range(emb // LANES):
                out_v[r, pl.ds(c * LANES, LANES)] = acc[c]

        for slot in range(NBUF):
            issue(slot, slot)

        def ring_body(g, carry):
            for slot in range(NBUF):
                r = NBUF * g + slot
                wait(r, slot)
                accumulate(r, slot)
                issue(r + NBUF, slot)
            return carry

        lax.fori_loop(0, nb // NBUF - 1, ring_body, 0)
        for slot in range(NBUF):
            r = nb - NBUF + slot
            wait(r, slot)
            accumulate(r, slot)

        pltpu.sync_copy(out_v, out_hbm.at[pl.ds(base, nb)])

    return pl.kernel(
        body,
        out_type=jax.ShapeDtypeStruct((B, emb), jnp.float32),
        mesh=mesh,
        scratch_types=[
            pltpu.VMEM((nb, 2, half), jnp.int32),
            pltpu.VMEM((NBUF, 2, half, emb), jnp.float32),
            pltpu.VMEM((nb, emb), jnp.float32),
        ]
        + [pltpu.SemaphoreType.DMA] * NBUF,
        compiler_params=pltpu.CompilerParams(use_tc_tiling_on_sc=False),
    )(data2, table)


def _tc_body(sums_ref, len_ref, w_ref, b_ref, out_ref):
    s = sums_ref[...]
    out = lax.dot_general(
        s, w_ref[...], (((1,), (1,)), ((), ())), preferred_element_type=jnp.float32
    )
    out_ref[...] = out / len_ref[...] + b_ref[...]


@jax.jit
def _tc_linear(sums, length_col, W, b_row):
    B = sums.shape[0]
    n_cls = W.shape[0]
    return pl.pallas_call(
        _tc_body,
        out_shape=jax.ShapeDtypeStruct((B, n_cls), jnp.float32),
    )(sums, length_col, W, b_row)


def kernel(data, length, table, W, b):
    B, L = data.shape
    emb = table.shape[1]
    half = L // 2
    assert L % 2 == 0 and half <= 128 and emb % LANES == 0
    nw = 32
    nb = B // nw
    data2 = data.astype(jnp.int32).reshape(B, 2, half)
    sums = _sc_pooled_sums(data2, table, nb=nb, half=half, emb=emb)
    length_col = length.astype(jnp.float32).reshape(B, 1)
    return _tc_linear(sums, length_col, W, b.reshape(1, -1))
